# Initial kernel scaffold; baseline (speedup 1.0000x reference)
import jax
import jax.numpy as jnp
from jax import lax
from jax.experimental import pallas as pl
from jax.experimental.pallas import tpu as pltpu
from jax.experimental.pallas import tpu_sc as plsc

N_USERS = 25000
N_ITEMS = 25000
N_NODES = N_USERS + N_ITEMS
DIM = 64
N_EDGES = 800000
N_LAYERS = 3
BATCH = 4096

NC = 2
NS = 16
LANES = 16

HALF = 25000
ACC_ROWS = 25024
PAD_ROWS = 2 * ACC_ROWS

E_PAD = 819200
EDGES_PER_TILE = E_PAD // NS
CHUNK = 1024
N_CHUNKS = EDGES_PER_TILE // CHUNK
NDMA = 8
DMA_E = CHUNK // NDMA

ZROWS = ACC_ROWS // NS

_mesh = plsc.VectorSubcoreMesh(core_axis_name="c", subcore_axis_name="s")


def _layer_body(table, src_e, dst_e, w_e, zeros_hbm, out,
                acc, src2d, dst2d, didx, wbuf, rows, sem, sem2):
  cid = lax.axis_index("c")
  sid = lax.axis_index("s")

  pltpu.sync_copy(zeros_hbm.at[pl.ds(0, ZROWS)],
                  acc.at[pl.ds(sid * ZROWS, ZROWS)])
  plsc.subcore_barrier()

  base_node = cid * HALF
  edge_base = sid * EDGES_PER_TILE

  def chunk_body(m, carry):
    off = edge_base + m * CHUNK
    for j in range(NDMA):
      pltpu.sync_copy(src_e.at[pl.ds(off + j * DMA_E, DMA_E)], src2d.at[j])
      pltpu.sync_copy(dst_e.at[pl.ds(off + j * DMA_E, DMA_E)], dst2d.at[j])
    pltpu.sync_copy(w_e.at[pl.ds(off, CHUNK)], wbuf)

    for j in range(NDMA):
      for g in range(DMA_E // LANES):
        d = dst2d[j, pl.ds(g * LANES, LANES)]
        local = d - base_node
        valid = (local >= 0) & (local < HALF)
        didx[j, pl.ds(g * LANES, LANES)] = jnp.where(valid, local, HALF)

    gathers = [
        pltpu.async_copy(table.at[src2d.at[j]],
                         rows.at[pl.ds(j * DMA_E, DMA_E)], sem)
        for j in range(NDMA)
    ]
    for g in gathers:
      g.wait()

    def scale_body(e, c):
      wv = plsc.load_gather(wbuf, [jnp.full((LANES,), e, jnp.int32)])
      for cb in range(DIM // LANES):
        rows[e, pl.ds(cb * LANES, LANES)] = (
            rows[e, pl.ds(cb * LANES, LANES)] * wv)
      return c
    lax.fori_loop(0, CHUNK, scale_body, 0)

    scatters = [
        pltpu.async_copy(rows.at[pl.ds(j * DMA_E, DMA_E)],
                         acc.at[didx.at[j]], sem2, add=True)
        for j in range(NDMA)
    ]
    for s in scatters:
      s.wait()
    return carry

  lax.fori_loop(0, N_CHUNKS, chunk_body, 0)
  plsc.subcore_barrier()

  ROWS_A = 1562
  pltpu.sync_copy(acc.at[pl.ds(sid * ROWS_A, ROWS_A)],
                  out.at[pl.ds(base_node + sid * ROWS_A, ROWS_A)])

  @pl.when(sid < HALF - NS * ROWS_A)
  def _():
    pltpu.sync_copy(acc.at[pl.ds(NS * ROWS_A + sid, 1)],
                    out.at[pl.ds(base_node + NS * ROWS_A + sid, 1)])


_layer = pl.kernel(
    _layer_body,
    out_type=jax.ShapeDtypeStruct((PAD_ROWS, DIM), jnp.float32),
    mesh=_mesh,
    scratch_types=[
        pltpu.VMEM_SHARED((ACC_ROWS, DIM), jnp.float32),
        pltpu.VMEM((NDMA, DMA_E), jnp.int32),
        pltpu.VMEM((NDMA, DMA_E), jnp.int32),
        pltpu.VMEM((NDMA, DMA_E), jnp.int32),
        pltpu.VMEM((CHUNK,), jnp.float32),
        pltpu.VMEM((CHUNK, DIM), jnp.float32),
        pltpu.SemaphoreType.DMA,
        pltpu.SemaphoreType.DMA,
    ],
)


B_PER_TILE = BATCH // (NC * NS)


def _combine_body(t0, t1, t2, t3, users, items, out,
                  uidx, iidx, urows, irows, gbuf, sem):
  cid = lax.axis_index("c")
  sid = lax.axis_index("s")
  wid = sid * NC + cid
  base = wid * B_PER_TILE

  pltpu.sync_copy(users.at[pl.ds(base, B_PER_TILE)], uidx.at[0])
  pltpu.sync_copy(items.at[pl.ds(base, B_PER_TILE)], iidx.at[0])
  for g in range(B_PER_TILE // LANES):
    iidx[0, pl.ds(g * LANES, LANES)] = (
        iidx[0, pl.ds(g * LANES, LANES)] + N_USERS)

  copies = []
  for k, t in enumerate((t0, t1, t2, t3)):
    copies.append(pltpu.async_copy(t.at[uidx.at[0]], urows.at[k], sem))
    copies.append(pltpu.async_copy(t.at[iidx.at[0]], irows.at[k], sem))
  for c in copies:
    c.wait()

  def dot_body(e, c):
    s = jnp.float32(0.0)
    for cb in range(DIM // LANES):
      sl = pl.ds(cb * LANES, LANES)
      us = (urows[0, e, sl] + urows[1, e, sl]
            + urows[2, e, sl] + urows[3, e, sl])
      vs = (irows[0, e, sl] + irows[1, e, sl]
            + irows[2, e, sl] + irows[3, e, sl])
      s = s + jnp.sum(us * vs, axis=0)
    gbuf[0, e] = s * jnp.float32(1.0 / ((N_LAYERS + 1) * (N_LAYERS + 1)))
    return c
  lax.fori_loop(0, B_PER_TILE, dot_body, 0)

  pltpu.sync_copy(gbuf.at[0], out.at[pl.ds(base, B_PER_TILE)])


_combine = pl.kernel(
    _combine_body,
    out_type=jax.ShapeDtypeStruct((BATCH,), jnp.float32),
    mesh=_mesh,
    scratch_types=[
        pltpu.VMEM((1, B_PER_TILE), jnp.int32),
        pltpu.VMEM((1, B_PER_TILE), jnp.int32),
        pltpu.VMEM((4, B_PER_TILE, DIM), jnp.float32),
        pltpu.VMEM((4, B_PER_TILE, DIM), jnp.float32),
        pltpu.VMEM((1, B_PER_TILE), jnp.float32),
        pltpu.SemaphoreType.DMA,
    ],
)


@jax.jit
def kernel(users, items, edge_index, edge_weight, user_emb, item_emb):
  src = edge_index[0]
  dst = edge_index[1]
  pad = E_PAD - N_EDGES
  src_p = jnp.concatenate([src, jnp.zeros((pad,), jnp.int32)])
  dst_p = jnp.concatenate([dst, jnp.full((pad,), N_NODES, jnp.int32)])
  w_p = jnp.concatenate([edge_weight, jnp.zeros((pad,), jnp.float32)])
  zeros_hbm = jnp.zeros((ZROWS, DIM), jnp.float32)

  t0 = jnp.concatenate(
      [user_emb, item_emb, jnp.zeros((PAD_ROWS - N_NODES, DIM), jnp.float32)],
      axis=0)
  t1 = _layer(t0, src_p, dst_p, w_p, zeros_hbm)
  t2 = _layer(t1, src_p, dst_p, w_p, zeros_hbm)
  t3 = _layer(t2, src_p, dst_p, w_p, zeros_hbm)
  return _combine(t0, t1, t2, t3, users, items)


# speedup vs baseline: 1.6722x; 1.6722x over previous
import jax
import jax.numpy as jnp
from jax import lax
from jax.experimental import pallas as pl
from jax.experimental.pallas import tpu as pltpu
from jax.experimental.pallas import tpu_sc as plsc

N_USERS = 25000
N_ITEMS = 25000
N_NODES = N_USERS + N_ITEMS
DIM = 64
N_EDGES = 800000
N_LAYERS = 3
BATCH = 4096

NC = 2
NS = 16
LANES = 16

HALF = 25000
ACC_ROWS = 25088
PAD_ROWS = 50048

E_PAD = 819200
EDGES_PER_TILE = E_PAD // NS
CHUNK = 256
N_CHUNKS = EDGES_PER_TILE // CHUNK
NDMA = 2
DMA_E = CHUNK // NDMA

ZROWS = ACC_ROWS // NS

_mesh = plsc.VectorSubcoreMesh(core_axis_name="c", subcore_axis_name="s")


def _layer_body(table, src_e, dst_e, w_e, zeros_hbm, out,
                acc, src2d, dst2d, didx, wbuf, rows, sem, sem2):
  cid = lax.axis_index("c")
  sid = lax.axis_index("s")

  pltpu.sync_copy(zeros_hbm.at[pl.ds(0, ZROWS)],
                  acc.at[pl.ds(sid * ZROWS, ZROWS)])
  plsc.subcore_barrier()

  base_node = cid * HALF
  edge_base = sid * EDGES_PER_TILE

  def chunk_body(m, carry):
    off = edge_base + m * CHUNK
    for j in range(NDMA):
      pltpu.sync_copy(src_e.at[pl.ds(off + j * DMA_E, DMA_E)], src2d.at[j])
      pltpu.sync_copy(dst_e.at[pl.ds(off + j * DMA_E, DMA_E)], dst2d.at[j])
    pltpu.sync_copy(w_e.at[pl.ds(off, CHUNK)], wbuf)

    for j in range(NDMA):
      for g in range(DMA_E // LANES):
        d = dst2d[j, pl.ds(g * LANES, LANES)]
        local = d - base_node
        valid = (local >= 0) & (local < HALF)
        didx[j, pl.ds(g * LANES, LANES)] = jnp.where(valid, local, HALF)

    gathers = [
        pltpu.async_copy(table.at[src2d.at[j]],
                         rows.at[pl.ds(j * DMA_E, DMA_E)], sem)
        for j in range(NDMA)
    ]
    for g in gathers:
      g.wait()

    def scale_body(g, c):
      w16 = wbuf[pl.ds(g * LANES, LANES)]
      for j in range(LANES):
        wv = jnp.full((LANES,), w16[j], jnp.float32)
        e = g * LANES + j
        for cb in range(DIM // LANES):
          rows[e, pl.ds(cb * LANES, LANES)] = (
              rows[e, pl.ds(cb * LANES, LANES)] * wv)
      return c
    lax.fori_loop(0, CHUNK // LANES, scale_body, 0)

    scatters = [
        pltpu.async_copy(rows.at[pl.ds(j * DMA_E, DMA_E)],
                         acc.at[didx.at[j]], sem2, add=True)
        for j in range(NDMA)
    ]
    for s in scatters:
      s.wait()
    return carry

  lax.fori_loop(0, N_CHUNKS, chunk_body, 0)
  plsc.subcore_barrier()

  ROWS_A = 1560
  pltpu.sync_copy(acc.at[pl.ds(sid * ROWS_A, ROWS_A)],
                  out.at[pl.ds(base_node + sid * ROWS_A, ROWS_A)])

  @pl.when(sid < (HALF - NS * ROWS_A) // 8)
  def _():
    pltpu.sync_copy(acc.at[pl.ds(NS * ROWS_A + sid * 8, 8)],
                    out.at[pl.ds(base_node + NS * ROWS_A + sid * 8, 8)])


_layer = pl.kernel(
    _layer_body,
    out_type=jax.ShapeDtypeStruct((PAD_ROWS, DIM), jnp.float32),
    mesh=_mesh,
    compiler_params=pltpu.CompilerParams(use_tc_tiling_on_sc=False, needs_layout_passes=False),
    scratch_types=[
        pltpu.VMEM_SHARED((ACC_ROWS, DIM), jnp.float32),
        pltpu.VMEM((NDMA, DMA_E), jnp.int32),
        pltpu.VMEM((NDMA, DMA_E), jnp.int32),
        pltpu.VMEM((NDMA, DMA_E), jnp.int32),
        pltpu.VMEM((CHUNK,), jnp.float32),
        pltpu.VMEM((CHUNK, DIM), jnp.float32),
        pltpu.SemaphoreType.DMA,
        pltpu.SemaphoreType.DMA,
    ],
)


B_PER_TILE = BATCH // (NC * NS)


def _combine_body(t0, t1, t2, t3, users, items, out,
                  uidx, iidx, urows, irows, gbuf, sem):
  cid = lax.axis_index("c")
  sid = lax.axis_index("s")
  wid = sid * NC + cid
  base = wid * B_PER_TILE

  pltpu.sync_copy(users.at[pl.ds(base, B_PER_TILE)], uidx.at[0])
  pltpu.sync_copy(items.at[pl.ds(base, B_PER_TILE)], iidx.at[0])
  for g in range(B_PER_TILE // LANES):
    iidx[0, pl.ds(g * LANES, LANES)] = (
        iidx[0, pl.ds(g * LANES, LANES)] + N_USERS)

  copies = []
  for k, t in enumerate((t0, t1, t2, t3)):
    copies.append(pltpu.async_copy(t.at[uidx.at[0]], urows.at[k], sem))
    copies.append(pltpu.async_copy(t.at[iidx.at[0]], irows.at[k], sem))
  for c in copies:
    c.wait()

  lanes = lax.iota(jnp.int32, LANES)

  def dot_body(g, c):
    res = jnp.zeros((LANES,), jnp.float32)
    for j in range(LANES):
      e = g * LANES + j
      p = jnp.zeros((LANES,), jnp.float32)
      for cb in range(DIM // LANES):
        sl = pl.ds(cb * LANES, LANES)
        us = (urows[0, e, sl] + urows[1, e, sl]
              + urows[2, e, sl] + urows[3, e, sl])
        vs = (irows[0, e, sl] + irows[1, e, sl]
              + irows[2, e, sl] + irows[3, e, sl])
        p = p + us * vs
      s = jnp.sum(p, axis=0) * jnp.float32(1.0 / ((N_LAYERS + 1) ** 2))
      res = jnp.where(lanes == j, jnp.full((LANES,), s, jnp.float32), res)
    gbuf[pl.ds(g * LANES, LANES)] = res
    return c
  lax.fori_loop(0, B_PER_TILE // LANES, dot_body, 0)

  pltpu.sync_copy(gbuf, out.at[pl.ds(base, B_PER_TILE)])


_combine = pl.kernel(
    _combine_body,
    out_type=jax.ShapeDtypeStruct((BATCH,), jnp.float32),
    mesh=_mesh,
    compiler_params=pltpu.CompilerParams(use_tc_tiling_on_sc=False, needs_layout_passes=False),
    scratch_types=[
        pltpu.VMEM((1, B_PER_TILE), jnp.int32),
        pltpu.VMEM((1, B_PER_TILE), jnp.int32),
        pltpu.VMEM((4, B_PER_TILE, DIM), jnp.float32),
        pltpu.VMEM((4, B_PER_TILE, DIM), jnp.float32),
        pltpu.VMEM((B_PER_TILE,), jnp.float32),
        pltpu.SemaphoreType.DMA,
    ],
)


@jax.jit
def kernel(users, items, edge_index, edge_weight, user_emb, item_emb):
  src = edge_index[0]
  dst = edge_index[1]
  pad = E_PAD - N_EDGES
  src_p = jnp.concatenate([src, jnp.zeros((pad,), jnp.int32)])
  dst_p = jnp.concatenate([dst, jnp.full((pad,), N_NODES, jnp.int32)])
  w_p = jnp.concatenate([edge_weight, jnp.zeros((pad,), jnp.float32)])
  zeros_hbm = jnp.zeros((ZROWS, DIM), jnp.float32)

  t0 = jnp.concatenate(
      [user_emb, item_emb, jnp.zeros((PAD_ROWS - N_NODES, DIM), jnp.float32)],
      axis=0)
  t1 = _layer(t0, src_p, dst_p, w_p, zeros_hbm)
  t2 = _layer(t1, src_p, dst_p, w_p, zeros_hbm)
  t3 = _layer(t2, src_p, dst_p, w_p, zeros_hbm)
  return _combine(t0, t1, t2, t3, users, items)


# speedup vs baseline: 2.5080x; 1.4999x over previous
import jax
import jax.numpy as jnp
from jax import lax
from jax.experimental import pallas as pl
from jax.experimental.pallas import tpu as pltpu
from jax.experimental.pallas import tpu_sc as plsc

N_USERS = 25000
N_ITEMS = 25000
N_NODES = N_USERS + N_ITEMS
DIM = 64
N_EDGES = 800000
N_LAYERS = 3
BATCH = 4096

NC = 2
NS = 16
LANES = 16

HALF = 25000
ACC_ROWS = 25088
PAD_ROWS = 50048

DMA_E = 128
NBUF = 3
BLK = NBUF * DMA_E
N_OUTER = 134
EDGES_PER_TILE = N_OUTER * BLK
E_PAD = NS * EDGES_PER_TILE
N_CHUNKS = N_OUTER * NBUF

ZROWS = ACC_ROWS // NS

_mesh = plsc.VectorSubcoreMesh(core_axis_name="c", subcore_axis_name="s")


def _layer_body(table, src_e, dst_e, w_e, zeros_hbm, out,
                acc, src_s, dst_s, w_s, didx, rows, sem_l, sem_g, sem_s):
  cid = lax.axis_index("c")
  sid = lax.axis_index("s")

  pltpu.sync_copy(zeros_hbm.at[pl.ds(0, ZROWS)],
                  acc.at[pl.ds(sid * ZROWS, ZROWS)])
  plsc.subcore_barrier()

  base_node = cid * HALF
  edge_base = sid * EDGES_PER_TILE

  def issue_loads(m, p):
    off = edge_base + m * BLK
    pltpu.async_copy(src_e.at[pl.ds(off, BLK)], src_s.at[p], sem_l)
    pltpu.async_copy(dst_e.at[pl.ds(off, BLK)], dst_s.at[p], sem_l)
    pltpu.async_copy(w_e.at[pl.ds(off, BLK)], w_s.at[p], sem_l)

  def wait_loads(p):
    pltpu.make_async_copy(src_e.at[pl.ds(0, BLK)], src_s.at[p], sem_l).wait()
    pltpu.make_async_copy(dst_e.at[pl.ds(0, BLK)], dst_s.at[p], sem_l).wait()
    pltpu.make_async_copy(w_e.at[pl.ds(0, BLK)], w_s.at[p], sem_l).wait()

  def transform(p, boff, bn):
    for g in range(DMA_E // LANES):
      d = dst_s[p, pl.ds(boff + g * LANES, LANES)]
      local = d - base_node
      valid = (local >= 0) & (local < HALF)
      didx[bn, pl.ds(g * LANES, LANES)] = jnp.where(valid, local, HALF)

  def issue_gather(p, boff, bn):
    pltpu.async_copy(table.at[src_s.at[p, pl.ds(boff, DMA_E)]],
                     rows.at[bn], sem_g)

  def wait_gather(bn):
    pltpu.make_async_copy(table.at[didx.at[bn]], rows.at[bn], sem_g).wait()

  def issue_scatter(bn):
    pltpu.async_copy(rows.at[bn], acc.at[didx.at[bn]], sem_s, add=True)

  def wait_scatter(bn):
    pltpu.make_async_copy(rows.at[bn], acc.at[didx.at[bn]], sem_s).wait()

  def scale(p, b):
    def scale_body(g, c):
      w16 = w_s[p, pl.ds(b * DMA_E + g * LANES, LANES)]
      for j in range(LANES):
        wv = jnp.full((LANES,), w16[j], jnp.float32)
        e = g * LANES + j
        for cb in range(DIM // LANES):
          rows[b, e, pl.ds(cb * LANES, LANES)] = (
              rows[b, e, pl.ds(cb * LANES, LANES)] * wv)
      return c
    lax.fori_loop(0, DMA_E // LANES, scale_body, 0)

  issue_loads(0, 0)
  wait_loads(0)
  transform(0, 0, 0)
  issue_gather(0, 0, 0)

  def outer(m, carry):
    p = lax.rem(m, 2)
    pn = 1 - p
    for b in range(NBUF):
      bn = (b + 1) % NBUF
      if b == 0:
        @pl.when(m + 1 < N_OUTER)
        def _():
          issue_loads(m + 1, pn)
      if b >= 2:
        wait_scatter(bn)
      else:
        @pl.when(m > 0)
        def _():
          wait_scatter(bn)
      if b < NBUF - 1:
        transform(p, (b + 1) * DMA_E, bn)
        issue_gather(p, (b + 1) * DMA_E, bn)
      else:
        @pl.when(m + 1 < N_OUTER)
        def _():
          wait_loads(pn)
          transform(pn, 0, bn)
          issue_gather(pn, 0, bn)
      wait_gather(b)
      scale(p, b)
      issue_scatter(b)
    return carry

  lax.fori_loop(0, N_OUTER, outer, 0)
  wait_scatter((N_CHUNKS - 2) % NBUF)
  wait_scatter((N_CHUNKS - 1) % NBUF)
  plsc.subcore_barrier()

  ROWS_A = 1560
  pltpu.sync_copy(acc.at[pl.ds(sid * ROWS_A, ROWS_A)],
                  out.at[pl.ds(base_node + sid * ROWS_A, ROWS_A)])

  @pl.when(sid < (HALF - NS * ROWS_A) // 8)
  def _():
    pltpu.sync_copy(acc.at[pl.ds(NS * ROWS_A + sid * 8, 8)],
                    out.at[pl.ds(base_node + NS * ROWS_A + sid * 8, 8)])


_layer = pl.kernel(
    _layer_body,
    out_type=jax.ShapeDtypeStruct((PAD_ROWS, DIM), jnp.float32),
    mesh=_mesh,
    compiler_params=pltpu.CompilerParams(use_tc_tiling_on_sc=False, needs_layout_passes=False),
    scratch_types=[
        pltpu.VMEM_SHARED((ACC_ROWS, DIM), jnp.float32),
        pltpu.VMEM((2, BLK), jnp.int32),
        pltpu.VMEM((2, BLK), jnp.int32),
        pltpu.VMEM((2, BLK), jnp.float32),
        pltpu.VMEM((NBUF, DMA_E), jnp.int32),
        pltpu.VMEM((NBUF, DMA_E, DIM), jnp.float32),
        pltpu.SemaphoreType.DMA,
        pltpu.SemaphoreType.DMA,
        pltpu.SemaphoreType.DMA,
    ],
)


B_PER_TILE = BATCH // (NC * NS)


def _combine_body(t0, t1, t2, t3, users, items, out,
                  uidx, iidx, urows, irows, gbuf, sem):
  cid = lax.axis_index("c")
  sid = lax.axis_index("s")
  wid = sid * NC + cid
  base = wid * B_PER_TILE

  pltpu.sync_copy(users.at[pl.ds(base, B_PER_TILE)], uidx.at[0])
  pltpu.sync_copy(items.at[pl.ds(base, B_PER_TILE)], iidx.at[0])
  for g in range(B_PER_TILE // LANES):
    iidx[0, pl.ds(g * LANES, LANES)] = (
        iidx[0, pl.ds(g * LANES, LANES)] + N_USERS)

  copies = []
  for k, t in enumerate((t0, t1, t2, t3)):
    copies.append(pltpu.async_copy(t.at[uidx.at[0]], urows.at[k], sem))
    copies.append(pltpu.async_copy(t.at[iidx.at[0]], irows.at[k], sem))
  for c in copies:
    c.wait()

  lanes = lax.iota(jnp.int32, LANES)

  def dot_body(g, c):
    res = jnp.zeros((LANES,), jnp.float32)
    for j in range(LANES):
      e = g * LANES + j
      p = jnp.zeros((LANES,), jnp.float32)
      for cb in range(DIM // LANES):
        sl = pl.ds(cb * LANES, LANES)
        us = (urows[0, e, sl] + urows[1, e, sl]
              + urows[2, e, sl] + urows[3, e, sl])
        vs = (irows[0, e, sl] + irows[1, e, sl]
              + irows[2, e, sl] + irows[3, e, sl])
        p = p + us * vs
      s = jnp.sum(p, axis=0) * jnp.float32(1.0 / ((N_LAYERS + 1) ** 2))
      res = jnp.where(lanes == j, jnp.full((LANES,), s, jnp.float32), res)
    gbuf[pl.ds(g * LANES, LANES)] = res
    return c
  lax.fori_loop(0, B_PER_TILE // LANES, dot_body, 0)

  pltpu.sync_copy(gbuf, out.at[pl.ds(base, B_PER_TILE)])


_combine = pl.kernel(
    _combine_body,
    out_type=jax.ShapeDtypeStruct((BATCH,), jnp.float32),
    mesh=_mesh,
    compiler_params=pltpu.CompilerParams(use_tc_tiling_on_sc=False, needs_layout_passes=False),
    scratch_types=[
        pltpu.VMEM((1, B_PER_TILE), jnp.int32),
        pltpu.VMEM((1, B_PER_TILE), jnp.int32),
        pltpu.VMEM((4, B_PER_TILE, DIM), jnp.float32),
        pltpu.VMEM((4, B_PER_TILE, DIM), jnp.float32),
        pltpu.VMEM((B_PER_TILE,), jnp.float32),
        pltpu.SemaphoreType.DMA,
    ],
)


@jax.jit
def kernel(users, items, edge_index, edge_weight, user_emb, item_emb):
  src = edge_index[0]
  dst = edge_index[1]
  pad = E_PAD - N_EDGES
  src_p = jnp.concatenate([src, jnp.zeros((pad,), jnp.int32)])
  dst_p = jnp.concatenate([dst, jnp.full((pad,), N_NODES, jnp.int32)])
  w_p = jnp.concatenate([edge_weight, jnp.zeros((pad,), jnp.float32)])
  zeros_hbm = jnp.zeros((ZROWS, DIM), jnp.float32)

  t0 = jnp.concatenate(
      [user_emb, item_emb, jnp.zeros((PAD_ROWS - N_NODES, DIM), jnp.float32)],
      axis=0)
  t1 = _layer(t0, src_p, dst_p, w_p, zeros_hbm)
  t2 = _layer(t1, src_p, dst_p, w_p, zeros_hbm)
  t3 = _layer(t2, src_p, dst_p, w_p, zeros_hbm)
  return _combine(t0, t1, t2, t3, users, items)


# speedup vs baseline: 4.0753x; 1.6249x over previous
import jax
import jax.numpy as jnp
from jax import lax
from jax.experimental import pallas as pl
from jax.experimental.pallas import tpu as pltpu
from jax.experimental.pallas import tpu_sc as plsc

N_USERS = 25000
N_ITEMS = 25000
N_NODES = N_USERS + N_ITEMS
DIM = 64
N_EDGES = 800000
N_LAYERS = 3
BATCH = 4096

NC = 2
NS = 16
NW = NC * NS
LANES = 16

HALF = 25000
ACC_ROWS = 25088
PAD_ROWS = 50048

DMA_E = 128
E_PAD = 823296
SCAN = E_PAD // NW
SBLK = 384
N_SBLK = SCAN // SBLK
GROUPS = SBLK // LANES
FB = 512
RSTRIDE = 26240
CAP = 1024

ZROWS = ACC_ROWS // NS

_mesh = plsc.VectorSubcoreMesh(core_axis_name="c", subcore_axis_name="s")
_params = pltpu.CompilerParams(use_tc_tiling_on_sc=False,
                               needs_layout_passes=False)


def _partition_body(src_e, dst_e, w_e, psrc, pdst, pw, pcnt,
                    st_s, st_d, st_w, cb_s, cb_d, cb_w, cnt_v,
                    sem_st, sem_f0, sem_f1):
  cid = lax.axis_index("c")
  sid = lax.axis_index("s")
  wid = sid * NC + cid
  ebase = wid * SCAN
  rbase = wid * RSTRIDE
  fsems = (sem_f0, sem_f1)

  def issue_stage(m, p):
    off = ebase + m * SBLK
    pltpu.async_copy(src_e.at[pl.ds(off, SBLK)], st_s.at[p], sem_st)
    pltpu.async_copy(dst_e.at[pl.ds(off, SBLK)], st_d.at[p], sem_st)
    pltpu.async_copy(w_e.at[pl.ds(off, SBLK)], st_w.at[p], sem_st)

  def wait_stage(p):
    pltpu.make_async_copy(src_e.at[pl.ds(0, SBLK)], st_s.at[p], sem_st).wait()
    pltpu.make_async_copy(dst_e.at[pl.ds(0, SBLK)], st_d.at[p], sem_st).wait()
    pltpu.make_async_copy(w_e.at[pl.ds(0, SBLK)], st_w.at[p], sem_st).wait()

  def flush_waits(q):
    pltpu.make_async_copy(cb_s.at[q, 0, pl.ds(0, FB)],
                          psrc.at[q, pl.ds(rbase, FB)], fsems[q]).wait()
    pltpu.make_async_copy(cb_d.at[q, 0, pl.ds(0, FB)],
                          pdst.at[q, pl.ds(rbase, FB)], fsems[q]).wait()
    pltpu.make_async_copy(cb_w.at[q, 0, pl.ds(0, FB)],
                          pw.at[q, pl.ds(rbase, FB)], fsems[q]).wait()

  def issue_flush(q, par, flush_idx):
    dst_off = rbase + flush_idx * FB
    pltpu.async_copy(cb_s.at[q, par, pl.ds(0, FB)],
                     psrc.at[q, pl.ds(dst_off, FB)], fsems[q])
    pltpu.async_copy(cb_d.at[q, par, pl.ds(0, FB)],
                     pdst.at[q, pl.ds(dst_off, FB)], fsems[q])
    pltpu.async_copy(cb_w.at[q, par, pl.ds(0, FB)],
                     pw.at[q, pl.ds(dst_off, FB)], fsems[q])

  issue_stage(0, 0)

  def block_body(m, carry):
    f0, f1, c0, c1 = carry
    p = lax.rem(m, 2)
    wait_stage(p)

    @pl.when(m + 1 < N_SBLK)
    def _():
      issue_stage(m + 1, 1 - p)

    def group_body(g, carry2):
      fs = list(carry2[:2])
      cs = list(carry2[2:])
      s = st_s[p, pl.ds(g * LANES, LANES)]
      d = st_d[p, pl.ds(g * LANES, LANES)]
      w = st_w[p, pl.ds(g * LANES, LANES)]
      l1 = d - HALF
      locals_ = (d, l1)
      valids = (d < HALF, (l1 >= 0) & (l1 < HALF))
      for q in range(NC):
        fq, cq = fs[q], cs[q]
        par = lax.rem(cq, 3)
        vq = valids[q]
        plsc.store_compressed(cb_s.at[q, par, pl.ds(fq, LANES)], s, mask=vq)
        plsc.store_compressed(cb_d.at[q, par, pl.ds(fq, LANES)],
                              locals_[q], mask=vq)
        plsc.store_compressed(cb_w.at[q, par, pl.ds(fq, LANES)], w, mask=vq)
        n = plsc.all_reduce_population_count(vq)[0]
        fq = fq + n
        full = fq >= FB

        @pl.when(full)
        def _(q=q, fq=fq, cq=cq, par=par):
          npar = lax.rem(cq + 1, 3)

          @pl.when(cq >= 2)
          def _():
            flush_waits(q)
          issue_flush(q, par, cq)
          cb_s[q, npar, pl.ds(0, LANES)] = cb_s[q, par, pl.ds(FB, LANES)]
          cb_d[q, npar, pl.ds(0, LANES)] = cb_d[q, par, pl.ds(FB, LANES)]
          cb_w[q, npar, pl.ds(0, LANES)] = cb_w[q, par, pl.ds(FB, LANES)]

        fs[q] = jnp.where(full, fq - FB, fq)
        cs[q] = jnp.where(full, cq + 1, cq)
      return (fs[0], fs[1], cs[0], cs[1])

    return lax.fori_loop(0, GROUPS, group_body, (f0, f1, c0, c1))

  z = jnp.int32(0)
  f0, f1, c0, c1 = lax.fori_loop(0, N_SBLK, block_body, (z, z, z, z))

  zero16 = jnp.zeros((LANES,), jnp.int32)
  half16 = jnp.full((LANES,), HALF, jnp.int32)
  w016 = jnp.zeros((LANES,), jnp.float32)
  for q, fq, cq in ((0, f0, c0), (1, f1, c1)):
    par = lax.rem(cq, 3)
    for k in range(FB // LANES):
      cb_s[q, par, pl.ds(fq + k * LANES, LANES)] = zero16
      cb_d[q, par, pl.ds(fq + k * LANES, LANES)] = half16
      cb_w[q, par, pl.ds(fq + k * LANES, LANES)] = w016

    @pl.when(cq >= 1)
    def _(q=q):
      flush_waits(q)

    @pl.when(cq >= 2)
    def _(q=q):
      flush_waits(q)
    issue_flush(q, par, cq)
    flush_waits(q)
    cnt_v[pl.ds(0, LANES)] = jnp.full((LANES,), (cq + 1) * (FB // DMA_E),
                                      jnp.int32)
    pltpu.sync_copy(cnt_v, pcnt.at[q * NW + wid])


_partition = pl.kernel(
    _partition_body,
    out_type=(
        jax.ShapeDtypeStruct((NC, NW * RSTRIDE), jnp.int32),
        jax.ShapeDtypeStruct((NC, NW * RSTRIDE), jnp.int32),
        jax.ShapeDtypeStruct((NC, NW * RSTRIDE), jnp.float32),
        jax.ShapeDtypeStruct((NC * NW, LANES), jnp.int32),
    ),
    mesh=_mesh,
    compiler_params=_params,
    scratch_types=[
        pltpu.VMEM((2, SBLK), jnp.int32),
        pltpu.VMEM((2, SBLK), jnp.int32),
        pltpu.VMEM((2, SBLK), jnp.float32),
        pltpu.VMEM((NC, 3, CAP), jnp.int32),
        pltpu.VMEM((NC, 3, CAP), jnp.int32),
        pltpu.VMEM((NC, 3, CAP), jnp.float32),
        pltpu.VMEM((LANES,), jnp.int32),
        pltpu.SemaphoreType.DMA,
        pltpu.SemaphoreType.DMA,
        pltpu.SemaphoreType.DMA,
    ],
)


CK = 64
RG = 6
RI = 8


def _layer_body(table, psrc, pdst, pw, pcnt, zeros_hbm, out,
                acc, csrc, cdst, cw, rows, cnt_v, sem_l, sem_g, sem_s):
  cid = lax.axis_index("c")
  sid = lax.axis_index("s")

  pltpu.sync_copy(zeros_hbm.at[pl.ds(0, ZROWS)],
                  acc.at[pl.ds(sid * ZROWS, ZROWS)])
  plsc.subcore_barrier()

  def run_region(region):
    ebase = region * RSTRIDE
    pltpu.sync_copy(pcnt.at[cid * NW + region], cnt_v)
    n = cnt_v[pl.ds(0, LANES)][0] * (DMA_E // CK)

    def issue_loads(c):
      off = ebase + c * CK
      slot = lax.rem(c, RI)
      pltpu.async_copy(psrc.at[cid, pl.ds(off, CK)], csrc.at[slot], sem_l)
      pltpu.async_copy(pdst.at[cid, pl.ds(off, CK)], cdst.at[slot], sem_l)
      pltpu.async_copy(pw.at[cid, pl.ds(off, CK)], cw.at[slot], sem_l)

    def wait_loads():
      pltpu.make_async_copy(psrc.at[cid, pl.ds(0, CK)],
                            csrc.at[0], sem_l).wait()
      pltpu.make_async_copy(pdst.at[cid, pl.ds(0, CK)],
                            cdst.at[0], sem_l).wait()
      pltpu.make_async_copy(pw.at[cid, pl.ds(0, CK)],
                            cw.at[0], sem_l).wait()

    def issue_gather(c):
      pltpu.async_copy(table.at[csrc.at[lax.rem(c, RI)]],
                       rows.at[lax.rem(c, RG)], sem_g)

    def wait_gather(c):
      pltpu.make_async_copy(table.at[csrc.at[lax.rem(c, RI)]],
                            rows.at[lax.rem(c, RG)], sem_g).wait()

    def issue_scatter(c):
      pltpu.async_copy(rows.at[lax.rem(c, RG)],
                       acc.at[cdst.at[lax.rem(c, RI)]], sem_s, add=True)

    def wait_scatter(c):
      pltpu.make_async_copy(rows.at[lax.rem(c, RG)],
                            acc.at[cdst.at[lax.rem(c, RI)]], sem_s).wait()

    def scale(c):
      b = lax.rem(c, RG)
      bw = lax.rem(c, RI)

      def scale_body(g, cc):
        w16 = cw[bw, pl.ds(g * LANES, LANES)]
        for j in range(LANES):
          wv = jnp.full((LANES,), w16[j], jnp.float32)
          e = g * LANES + j
          for cb in range(DIM // LANES):
            rows[b, e, pl.ds(cb * LANES, LANES)] = (
                rows[b, e, pl.ds(cb * LANES, LANES)] * wv)
        return cc
      lax.fori_loop(0, CK // LANES, scale_body, 0)

    for k in range(5):
      issue_loads(k)
    for k in range(3):
      wait_loads()
      issue_gather(k)

    def step(c, carry):
      @pl.when(c >= 3)
      def _():
        wait_scatter(c - 3)

      @pl.when(c + 5 < n)
      def _():
        issue_loads(c + 5)

      @pl.when(c + 3 < n)
      def _():
        wait_loads()
        issue_gather(c + 3)
      wait_gather(c)
      scale(c)
      issue_scatter(c)
      return carry

    lax.fori_loop(0, n, step, 0)
    wait_scatter(n - 3)
    wait_scatter(n - 2)
    wait_scatter(n - 1)

  run_region(2 * sid)
  run_region(2 * sid + 1)
  plsc.subcore_barrier()

  base_node = cid * HALF
  ROWS_A = 1560
  pltpu.sync_copy(acc.at[pl.ds(sid * ROWS_A, ROWS_A)],
                  out.at[pl.ds(base_node + sid * ROWS_A, ROWS_A)])

  @pl.when(sid < (HALF - NS * ROWS_A) // 8)
  def _():
    pltpu.sync_copy(acc.at[pl.ds(NS * ROWS_A + sid * 8, 8)],
                    out.at[pl.ds(base_node + NS * ROWS_A + sid * 8, 8)])


_layer = pl.kernel(
    _layer_body,
    out_type=jax.ShapeDtypeStruct((PAD_ROWS, DIM), jnp.float32),
    mesh=_mesh,
    compiler_params=_params,
    scratch_types=[
        pltpu.VMEM_SHARED((ACC_ROWS, DIM), jnp.float32),
        pltpu.VMEM((RI, CK), jnp.int32),
        pltpu.VMEM((RI, CK), jnp.int32),
        pltpu.VMEM((RI, CK), jnp.float32),
        pltpu.VMEM((RG, CK, DIM), jnp.float32),
        pltpu.VMEM((LANES,), jnp.int32),
        pltpu.SemaphoreType.DMA,
        pltpu.SemaphoreType.DMA,
        pltpu.SemaphoreType.DMA,
    ],
)


B_PER_TILE = BATCH // NW


def _combine_body(t0, t1, t2, t3, users, items, out,
                  uidx, iidx, urows, irows, gbuf, sem):
  cid = lax.axis_index("c")
  sid = lax.axis_index("s")
  wid = sid * NC + cid
  base = wid * B_PER_TILE

  pltpu.sync_copy(users.at[pl.ds(base, B_PER_TILE)], uidx.at[0])
  pltpu.sync_copy(items.at[pl.ds(base, B_PER_TILE)], iidx.at[0])
  for g in range(B_PER_TILE // LANES):
    iidx[0, pl.ds(g * LANES, LANES)] = (
        iidx[0, pl.ds(g * LANES, LANES)] + N_USERS)

  copies = []
  for k, t in enumerate((t0, t1, t2, t3)):
    copies.append(pltpu.async_copy(t.at[uidx.at[0]], urows.at[k], sem))
    copies.append(pltpu.async_copy(t.at[iidx.at[0]], irows.at[k], sem))
  for c in copies:
    c.wait()

  lanes = lax.iota(jnp.int32, LANES)

  def dot_body(g, c):
    res = jnp.zeros((LANES,), jnp.float32)
    for j in range(LANES):
      e = g * LANES + j
      p = jnp.zeros((LANES,), jnp.float32)
      for cb in range(DIM // LANES):
        sl = pl.ds(cb * LANES, LANES)
        us = (urows[0, e, sl] + urows[1, e, sl]
              + urows[2, e, sl] + urows[3, e, sl])
        vs = (irows[0, e, sl] + irows[1, e, sl]
              + irows[2, e, sl] + irows[3, e, sl])
        p = p + us * vs
      s = jnp.sum(p, axis=0) * jnp.float32(1.0 / ((N_LAYERS + 1) ** 2))
      res = jnp.where(lanes == j, jnp.full((LANES,), s, jnp.float32), res)
    gbuf[pl.ds(g * LANES, LANES)] = res
    return c
  lax.fori_loop(0, B_PER_TILE // LANES, dot_body, 0)

  pltpu.sync_copy(gbuf, out.at[pl.ds(base, B_PER_TILE)])


_combine = pl.kernel(
    _combine_body,
    out_type=jax.ShapeDtypeStruct((BATCH,), jnp.float32),
    mesh=_mesh,
    compiler_params=_params,
    scratch_types=[
        pltpu.VMEM((1, B_PER_TILE), jnp.int32),
        pltpu.VMEM((1, B_PER_TILE), jnp.int32),
        pltpu.VMEM((4, B_PER_TILE, DIM), jnp.float32),
        pltpu.VMEM((4, B_PER_TILE, DIM), jnp.float32),
        pltpu.VMEM((B_PER_TILE,), jnp.float32),
        pltpu.SemaphoreType.DMA,
    ],
)


@jax.jit
def kernel(users, items, edge_index, edge_weight, user_emb, item_emb):
  src = edge_index[0]
  dst = edge_index[1]
  pad = E_PAD - N_EDGES
  src_p = jnp.concatenate([src, jnp.zeros((pad,), jnp.int32)])
  dst_p = jnp.concatenate([dst, jnp.full((pad,), N_NODES, jnp.int32)])
  w_p = jnp.concatenate([edge_weight, jnp.zeros((pad,), jnp.float32)])
  zeros_hbm = jnp.zeros((ZROWS, DIM), jnp.float32)

  psrc, pdst, pw, pcnt = _partition(src_p, dst_p, w_p)

  t0 = jnp.concatenate(
      [user_emb, item_emb, jnp.zeros((PAD_ROWS - N_NODES, DIM), jnp.float32)],
      axis=0)
  t1 = _layer(t0, psrc, pdst, pw, pcnt, zeros_hbm)
  t2 = _layer(t1, psrc, pdst, pw, pcnt, zeros_hbm)
  t3 = _layer(t2, psrc, pdst, pw, pcnt, zeros_hbm)
  return _combine(t0, t1, t2, t3, users, items)


# speedup vs baseline: 4.1240x; 1.0119x over previous
import jax
import jax.numpy as jnp
from jax import lax
from jax.experimental import pallas as pl
from jax.experimental.pallas import tpu as pltpu
from jax.experimental.pallas import tpu_sc as plsc

N_USERS = 25000
N_ITEMS = 25000
N_NODES = N_USERS + N_ITEMS
DIM = 64
N_EDGES = 800000
N_LAYERS = 3
BATCH = 4096

NC = 2
NS = 16
NW = NC * NS
LANES = 16

HALF = 25000
ACC_ROWS = 25088
PAD_ROWS = 50048

DMA_E = 128
E_PAD = 823296
SCAN = E_PAD // NW
SBLK = 384
N_SBLK = SCAN // SBLK
GROUPS = SBLK // LANES
FB = 512
RF = 52
CAP = 1024

ZROWS = ACC_ROWS // NS

_mesh = plsc.VectorSubcoreMesh(core_axis_name="c", subcore_axis_name="s")
_params = pltpu.CompilerParams(use_tc_tiling_on_sc=False,
                               needs_layout_passes=False)


def _partition_body(src_e, dst_e, w_e, ped, pcnt,
                    st_s, st_d, st_w, cb_p, cnt_v,
                    sem_st, sem_f0, sem_f1):
  cid = lax.axis_index("c")
  sid = lax.axis_index("s")
  wid = sid * NC + cid
  ebase = wid * SCAN
  fbase = wid * RF
  fsems = (sem_f0, sem_f1)

  def issue_stage(m, p):
    off = ebase + m * SBLK
    pltpu.async_copy(src_e.at[pl.ds(off, SBLK)], st_s.at[p], sem_st)
    pltpu.async_copy(dst_e.at[pl.ds(off, SBLK)], st_d.at[p], sem_st)
    pltpu.async_copy(w_e.at[pl.ds(off, SBLK)], st_w.at[p], sem_st)

  def wait_stage(p):
    pltpu.make_async_copy(src_e.at[pl.ds(0, SBLK)], st_s.at[p], sem_st).wait()
    pltpu.make_async_copy(dst_e.at[pl.ds(0, SBLK)], st_d.at[p], sem_st).wait()
    pltpu.make_async_copy(w_e.at[pl.ds(0, SBLK)], st_w.at[p], sem_st).wait()

  def flush_waits(q):
    pltpu.make_async_copy(cb_p.at[q, 0, pl.ds(0, 3), pl.ds(0, FB)],
                          ped.at[q, fbase], fsems[q]).wait()

  def issue_flush(q, par, flush_idx):
    pltpu.async_copy(cb_p.at[q, par, pl.ds(0, 3), pl.ds(0, FB)],
                     ped.at[q, fbase + flush_idx], fsems[q])

  issue_stage(0, 0)

  def block_body(m, carry):
    f0, f1, c0, c1 = carry
    p = lax.rem(m, 2)
    wait_stage(p)

    @pl.when(m + 1 < N_SBLK)
    def _():
      issue_stage(m + 1, 1 - p)

    def group_body(g, carry2):
      fs = list(carry2[:2])
      cs = list(carry2[2:])
      s = st_s[p, pl.ds(g * LANES, LANES)]
      d = st_d[p, pl.ds(g * LANES, LANES)]
      w = st_w[p, pl.ds(g * LANES, LANES)]
      wi = plsc.bitcast(w, jnp.int32)
      l1 = d - HALF
      locals_ = (d, l1)
      valids = (d < HALF, (l1 >= 0) & (l1 < HALF))
      for q in range(NC):
        fq, cq = fs[q], cs[q]
        par = lax.rem(cq, 3)
        vq = valids[q]
        plsc.store_compressed(cb_p.at[q, par, 0, pl.ds(fq, LANES)], s,
                              mask=vq)
        plsc.store_compressed(cb_p.at[q, par, 1, pl.ds(fq, LANES)],
                              locals_[q], mask=vq)
        plsc.store_compressed(cb_p.at[q, par, 2, pl.ds(fq, LANES)], wi,
                              mask=vq)
        n = plsc.all_reduce_population_count(vq)[0]
        fq = fq + n
        full = fq >= FB

        @pl.when(full)
        def _(q=q, fq=fq, cq=cq, par=par):
          npar = lax.rem(cq + 1, 3)

          @pl.when(cq >= 2)
          def _():
            flush_waits(q)
          issue_flush(q, par, cq)
          for r in range(3):
            cb_p[q, npar, r, pl.ds(0, LANES)] = (
                cb_p[q, par, r, pl.ds(FB, LANES)])

        fs[q] = jnp.where(full, fq - FB, fq)
        cs[q] = jnp.where(full, cq + 1, cq)
      return (fs[0], fs[1], cs[0], cs[1])

    return lax.fori_loop(0, GROUPS, group_body, (f0, f1, c0, c1))

  z = jnp.int32(0)
  f0, f1, c0, c1 = lax.fori_loop(0, N_SBLK, block_body, (z, z, z, z))

  zero16 = jnp.zeros((LANES,), jnp.int32)
  half16 = jnp.full((LANES,), HALF, jnp.int32)
  for q, fq, cq in ((0, f0, c0), (1, f1, c1)):
    par = lax.rem(cq, 3)
    for k in range(FB // LANES):
      cb_p[q, par, 0, pl.ds(fq + k * LANES, LANES)] = zero16
      cb_p[q, par, 1, pl.ds(fq + k * LANES, LANES)] = half16
      cb_p[q, par, 2, pl.ds(fq + k * LANES, LANES)] = zero16

    @pl.when(cq >= 1)
    def _(q=q):
      flush_waits(q)

    @pl.when(cq >= 2)
    def _(q=q):
      flush_waits(q)
    issue_flush(q, par, cq)
    flush_waits(q)
    cnt_v[pl.ds(0, LANES)] = jnp.full((LANES,), cq + 1, jnp.int32)
    pltpu.sync_copy(cnt_v, pcnt.at[q * NW + wid])


_partition = pl.kernel(
    _partition_body,
    out_type=(
        jax.ShapeDtypeStruct((NC, NW * RF, 3, FB), jnp.int32),
        jax.ShapeDtypeStruct((NC * NW, LANES), jnp.int32),
    ),
    mesh=_mesh,
    compiler_params=_params,
    scratch_types=[
        pltpu.VMEM((2, SBLK), jnp.int32),
        pltpu.VMEM((2, SBLK), jnp.int32),
        pltpu.VMEM((2, SBLK), jnp.float32),
        pltpu.VMEM((NC, 3, 3, CAP), jnp.int32),
        pltpu.VMEM((LANES,), jnp.int32),
        pltpu.SemaphoreType.DMA,
        pltpu.SemaphoreType.DMA,
        pltpu.SemaphoreType.DMA,
    ],
)


CK = 64
CPF = FB // CK
RG = 7
RI = 8


def _layer_body(table, ped, pcnt, zeros_hbm, out,
                acc, cidx, rows, cnt_v, sem_l, sem_g, sem_s):
  cid = lax.axis_index("c")
  sid = lax.axis_index("s")

  pltpu.sync_copy(zeros_hbm.at[pl.ds(0, ZROWS)],
                  acc.at[pl.ds(sid * ZROWS, ZROWS)])
  plsc.subcore_barrier()

  def run_region(region):
    fbase = region * RF
    pltpu.sync_copy(pcnt.at[cid * NW + region], cnt_v)
    n = cnt_v[pl.ds(0, LANES)][0] * CPF

    def issue_load(c):
      fl = fbase + lax.div(c, CPF)
      o = lax.rem(c, CPF) * CK
      pltpu.async_copy(ped.at[cid, fl, pl.ds(0, 3), pl.ds(o, CK)],
                       cidx.at[lax.rem(c, RI)], sem_l)

    def wait_load():
      pltpu.make_async_copy(ped.at[cid, fbase, pl.ds(0, 3), pl.ds(0, CK)],
                            cidx.at[0], sem_l).wait()

    def issue_gather(c):
      pltpu.async_copy(table.at[cidx.at[lax.rem(c, RI), 0]],
                       rows.at[lax.rem(c, RG)], sem_g)

    def wait_gather(c):
      pltpu.make_async_copy(table.at[cidx.at[lax.rem(c, RI), 0]],
                            rows.at[lax.rem(c, RG)], sem_g).wait()

    def issue_scatter(c):
      pltpu.async_copy(rows.at[lax.rem(c, RG)],
                       acc.at[cidx.at[lax.rem(c, RI), 1]], sem_s, add=True)

    def wait_scatter(c):
      pltpu.make_async_copy(rows.at[lax.rem(c, RG)],
                            acc.at[cidx.at[lax.rem(c, RI), 1]], sem_s).wait()

    def scale(c):
      b = lax.rem(c, RG)
      bw = lax.rem(c, RI)

      def scale_body(g, cc):
        w16 = plsc.bitcast(cidx[bw, 2, pl.ds(g * LANES, LANES)], jnp.float32)
        for j in range(LANES):
          wv = jnp.full((LANES,), w16[j], jnp.float32)
          e = g * LANES + j
          for cb in range(DIM // LANES):
            rows[b, e, pl.ds(cb * LANES, LANES)] = (
                rows[b, e, pl.ds(cb * LANES, LANES)] * wv)
        return cc
      lax.fori_loop(0, CK // LANES, scale_body, 0)

    for k in range(6):
      issue_load(k)
    for k in range(4):
      wait_load()
      issue_gather(k)

    def step(c, carry):
      @pl.when(c >= 2)
      def _():
        wait_scatter(c - 2)

      @pl.when(c + 6 < n)
      def _():
        issue_load(c + 6)

      @pl.when(c + 4 < n)
      def _():
        wait_load()
        issue_gather(c + 4)
      wait_gather(c)
      scale(c)
      issue_scatter(c)
      return carry

    lax.fori_loop(0, n, step, 0)
    wait_scatter(n - 2)
    wait_scatter(n - 1)

  run_region(2 * sid)
  run_region(2 * sid + 1)
  plsc.subcore_barrier()

  base_node = cid * HALF
  ROWS_A = 1560
  pltpu.sync_copy(acc.at[pl.ds(sid * ROWS_A, ROWS_A)],
                  out.at[pl.ds(base_node + sid * ROWS_A, ROWS_A)])

  @pl.when(sid < (HALF - NS * ROWS_A) // 8)
  def _():
    pltpu.sync_copy(acc.at[pl.ds(NS * ROWS_A + sid * 8, 8)],
                    out.at[pl.ds(base_node + NS * ROWS_A + sid * 8, 8)])


_layer = pl.kernel(
    _layer_body,
    out_type=jax.ShapeDtypeStruct((PAD_ROWS, DIM), jnp.float32),
    mesh=_mesh,
    compiler_params=_params,
    scratch_types=[
        pltpu.VMEM_SHARED((ACC_ROWS, DIM), jnp.float32),
        pltpu.VMEM((RI, 3, CK), jnp.int32),
        pltpu.VMEM((RG, CK, DIM), jnp.float32),
        pltpu.VMEM((LANES,), jnp.int32),
        pltpu.SemaphoreType.DMA,
        pltpu.SemaphoreType.DMA,
        pltpu.SemaphoreType.DMA,
    ],
)


B_PER_TILE = BATCH // NW


def _combine_body(t0, t1, t2, t3, users, items, out,
                  uidx, iidx, urows, irows, gbuf, sem):
  cid = lax.axis_index("c")
  sid = lax.axis_index("s")
  wid = sid * NC + cid
  base = wid * B_PER_TILE

  pltpu.sync_copy(users.at[pl.ds(base, B_PER_TILE)], uidx.at[0])
  pltpu.sync_copy(items.at[pl.ds(base, B_PER_TILE)], iidx.at[0])
  for g in range(B_PER_TILE // LANES):
    iidx[0, pl.ds(g * LANES, LANES)] = (
        iidx[0, pl.ds(g * LANES, LANES)] + N_USERS)

  copies = []
  for k, t in enumerate((t0, t1, t2, t3)):
    copies.append(pltpu.async_copy(t.at[uidx.at[0]], urows.at[k], sem))
    copies.append(pltpu.async_copy(t.at[iidx.at[0]], irows.at[k], sem))
  for c in copies:
    c.wait()

  lanes = lax.iota(jnp.int32, LANES)

  def dot_body(g, c):
    res = jnp.zeros((LANES,), jnp.float32)
    for j in range(LANES):
      e = g * LANES + j
      p = jnp.zeros((LANES,), jnp.float32)
      for cb in range(DIM // LANES):
        sl = pl.ds(cb * LANES, LANES)
        us = (urows[0, e, sl] + urows[1, e, sl]
              + urows[2, e, sl] + urows[3, e, sl])
        vs = (irows[0, e, sl] + irows[1, e, sl]
              + irows[2, e, sl] + irows[3, e, sl])
        p = p + us * vs
      s = jnp.sum(p, axis=0) * jnp.float32(1.0 / ((N_LAYERS + 1) ** 2))
      res = jnp.where(lanes == j, jnp.full((LANES,), s, jnp.float32), res)
    gbuf[pl.ds(g * LANES, LANES)] = res
    return c
  lax.fori_loop(0, B_PER_TILE // LANES, dot_body, 0)

  pltpu.sync_copy(gbuf, out.at[pl.ds(base, B_PER_TILE)])


_combine = pl.kernel(
    _combine_body,
    out_type=jax.ShapeDtypeStruct((BATCH,), jnp.float32),
    mesh=_mesh,
    compiler_params=_params,
    scratch_types=[
        pltpu.VMEM((1, B_PER_TILE), jnp.int32),
        pltpu.VMEM((1, B_PER_TILE), jnp.int32),
        pltpu.VMEM((4, B_PER_TILE, DIM), jnp.float32),
        pltpu.VMEM((4, B_PER_TILE, DIM), jnp.float32),
        pltpu.VMEM((B_PER_TILE,), jnp.float32),
        pltpu.SemaphoreType.DMA,
    ],
)


@jax.jit
def kernel(users, items, edge_index, edge_weight, user_emb, item_emb):
  src = edge_index[0]
  dst = edge_index[1]
  pad = E_PAD - N_EDGES
  src_p = jnp.concatenate([src, jnp.zeros((pad,), jnp.int32)])
  dst_p = jnp.concatenate([dst, jnp.full((pad,), N_NODES, jnp.int32)])
  w_p = jnp.concatenate([edge_weight, jnp.zeros((pad,), jnp.float32)])
  zeros_hbm = jnp.zeros((ZROWS, DIM), jnp.float32)

  ped, pcnt = _partition(src_p, dst_p, w_p)

  t0 = jnp.concatenate(
      [user_emb, item_emb, jnp.zeros((PAD_ROWS - N_NODES, DIM), jnp.float32)],
      axis=0)
  t1 = _layer(t0, ped, pcnt, zeros_hbm)
  t2 = _layer(t1, ped, pcnt, zeros_hbm)
  t3 = _layer(t2, ped, pcnt, zeros_hbm)
  return _combine(t0, t1, t2, t3, users, items)


# speedup vs baseline: 5.7319x; 1.3899x over previous
import jax
import jax.numpy as jnp
from jax import lax
from jax.experimental import pallas as pl
from jax.experimental.pallas import tpu as pltpu
from jax.experimental.pallas import tpu_sc as plsc

N_USERS = 25000
N_ITEMS = 25000
N_NODES = N_USERS + N_ITEMS
DIM = 64
N_EDGES = 800000
N_LAYERS = 3
BATCH = 4096

NC = 2
NS = 16
NW = NC * NS
LANES = 16

HALF = 25000
ACC_ROWS = 25088
PAD_ROWS = 50048

DMA_E = 128
E_PAD = 823296
SCAN = E_PAD // NW
SBLK = 384
N_SBLK = SCAN // SBLK
GROUPS = SBLK // LANES
FB = 512
RF = 52
CAP = 1024

ZROWS = ACC_ROWS // NS

_mesh = plsc.VectorSubcoreMesh(core_axis_name="c", subcore_axis_name="s")
_params = pltpu.CompilerParams(use_tc_tiling_on_sc=False,
                               needs_layout_passes=False)


def _pack_rows(src_f32, dst_i32, nrows):
  def row_body(r, c):
    for k in range(2):
      lo = plsc.bitcast(src_f32[r, pl.ds(k * LANES, LANES)], jnp.int32)
      hi = plsc.bitcast(src_f32[r, pl.ds((k + 2) * LANES, LANES)], jnp.int32)
      lo = jax.lax.shift_right_logical(lo + jnp.int32(0x8000), jnp.int32(16))
      hi = (hi + jnp.int32(0x8000)) & jnp.int32(-65536)
      dst_i32[r, pl.ds(k * LANES, LANES)] = lo | hi
    return c
  lax.fori_loop(0, nrows, row_body, 0)


def _partition_body(src_e, dst_e, w_e, t0, ped, pcnt, t0_bf,
                    st_s, st_d, st_w, cb_p, cnt_v, rowf, rowp,
                    sem_st, sem_f0, sem_f1, sem_t):
  cid = lax.axis_index("c")
  sid = lax.axis_index("s")
  wid = sid * NC + cid
  ebase = wid * SCAN
  fbase = wid * RF
  fsems = (sem_f0, sem_f1)

  def issue_stage(m, p):
    off = ebase + m * SBLK
    pltpu.async_copy(src_e.at[pl.ds(off, SBLK)], st_s.at[p], sem_st)
    pltpu.async_copy(dst_e.at[pl.ds(off, SBLK)], st_d.at[p], sem_st)
    pltpu.async_copy(w_e.at[pl.ds(off, SBLK)], st_w.at[p], sem_st)

  def wait_stage(p):
    pltpu.make_async_copy(src_e.at[pl.ds(0, SBLK)], st_s.at[p], sem_st).wait()
    pltpu.make_async_copy(dst_e.at[pl.ds(0, SBLK)], st_d.at[p], sem_st).wait()
    pltpu.make_async_copy(w_e.at[pl.ds(0, SBLK)], st_w.at[p], sem_st).wait()

  def flush_waits(q):
    pltpu.make_async_copy(cb_p.at[q, 0, pl.ds(0, 3), pl.ds(0, FB)],
                          ped.at[q, fbase], fsems[q]).wait()

  def issue_flush(q, par, flush_idx):
    pltpu.async_copy(cb_p.at[q, par, pl.ds(0, 3), pl.ds(0, FB)],
                     ped.at[q, fbase + flush_idx], fsems[q])

  TROWS = 1560
  tb = wid * TROWS

  issue_stage(0, 0)

  def block_body(m, carry):
    f0, f1, c0, c1 = carry
    p = lax.rem(m, 2)
    wait_stage(p)

    @pl.when(m + 1 < N_SBLK)
    def _():
      issue_stage(m + 1, 1 - p)

    def group_body(g, carry2):
      fs = list(carry2[:2])
      cs = list(carry2[2:])
      s = st_s[p, pl.ds(g * LANES, LANES)]
      d = st_d[p, pl.ds(g * LANES, LANES)]
      w = st_w[p, pl.ds(g * LANES, LANES)]
      wi = plsc.bitcast(w, jnp.int32)
      l1 = d - HALF
      locals_ = (d, l1)
      valids = (d < HALF, (l1 >= 0) & (l1 < HALF))
      for q in range(NC):
        fq, cq = fs[q], cs[q]
        par = lax.rem(cq, 3)
        vq = valids[q]
        plsc.store_compressed(cb_p.at[q, par, 0, pl.ds(fq, LANES)], s,
                              mask=vq)
        plsc.store_compressed(cb_p.at[q, par, 1, pl.ds(fq, LANES)],
                              locals_[q], mask=vq)
        plsc.store_compressed(cb_p.at[q, par, 2, pl.ds(fq, LANES)], wi,
                              mask=vq)
        n = plsc.all_reduce_population_count(vq)[0]
        fq = fq + n
        full = fq >= FB

        @pl.when(full)
        def _(q=q, fq=fq, cq=cq, par=par):
          npar = lax.rem(cq + 1, 3)

          @pl.when(cq >= 2)
          def _():
            flush_waits(q)
          issue_flush(q, par, cq)
          for r in range(3):
            cb_p[q, npar, r, pl.ds(0, LANES)] = (
                cb_p[q, par, r, pl.ds(FB, LANES)])

        fs[q] = jnp.where(full, fq - FB, fq)
        cs[q] = jnp.where(full, cq + 1, cq)
      return (fs[0], fs[1], cs[0], cs[1])

    return lax.fori_loop(0, GROUPS, group_body, (f0, f1, c0, c1))

  z = jnp.int32(0)
  f0, f1, c0, c1 = lax.fori_loop(0, N_SBLK, block_body, (z, z, z, z))

  zero16 = jnp.zeros((LANES,), jnp.int32)
  half16 = jnp.full((LANES,), HALF, jnp.int32)
  for q, fq, cq in ((0, f0, c0), (1, f1, c1)):
    par = lax.rem(cq, 3)
    for k in range(FB // LANES):
      cb_p[q, par, 0, pl.ds(fq + k * LANES, LANES)] = zero16
      cb_p[q, par, 1, pl.ds(fq + k * LANES, LANES)] = half16
      cb_p[q, par, 2, pl.ds(fq + k * LANES, LANES)] = zero16

    @pl.when(cq >= 1)
    def _(q=q):
      flush_waits(q)

    @pl.when(cq >= 2)
    def _(q=q):
      flush_waits(q)
    issue_flush(q, par, cq)
    flush_waits(q)
    cnt_v[pl.ds(0, LANES)] = jnp.full((LANES,), cq + 1, jnp.int32)
    pltpu.sync_copy(cnt_v, pcnt.at[q * NW + wid])

  def pack_blk(i, c):
    r0 = tb + i * 120
    pltpu.sync_copy(t0.at[pl.ds(r0, 120)], rowf)
    _pack_rows(rowf, rowp, 120)
    pltpu.sync_copy(rowp.at[pl.ds(0, 120)], t0_bf.at[pl.ds(r0, 120)])
    return c
  lax.fori_loop(0, TROWS // 120, pack_blk, 0)

  @pl.when(wid < 16)
  def _():
    r0 = NW * TROWS + wid * 8
    pltpu.sync_copy(t0.at[pl.ds(r0, 8)], rowf.at[pl.ds(0, 8)])
    _pack_rows(rowf, rowp, 8)
    pltpu.sync_copy(rowp.at[pl.ds(0, 8)], t0_bf.at[pl.ds(r0, 8)])


_partition = pl.kernel(
    _partition_body,
    out_type=(
        jax.ShapeDtypeStruct((NC, NW * RF, 3, FB), jnp.int32),
        jax.ShapeDtypeStruct((NC * NW, LANES), jnp.int32),
        jax.ShapeDtypeStruct((PAD_ROWS, DIM // 2), jnp.int32),
    ),
    mesh=_mesh,
    compiler_params=_params,
    scratch_types=[
        pltpu.VMEM((2, SBLK), jnp.int32),
        pltpu.VMEM((2, SBLK), jnp.int32),
        pltpu.VMEM((2, SBLK), jnp.float32),
        pltpu.VMEM((NC, 3, 3, CAP), jnp.int32),
        pltpu.VMEM((LANES,), jnp.int32),
        pltpu.VMEM((120, DIM), jnp.float32),
        pltpu.VMEM((120, DIM // 2), jnp.int32),
        pltpu.SemaphoreType.DMA,
        pltpu.SemaphoreType.DMA,
        pltpu.SemaphoreType.DMA,
        pltpu.SemaphoreType.DMA,
    ],
)


CK = 64
CPF = FB // CK
RI = 8


def _layer_body(table_bf, ped, pcnt, zeros_hbm, out, out_bf,
                acc, cidx, rows_bf, srows, pf, pi, cnt_v,
                sem_l, sem_g, sem_s):
  cid = lax.axis_index("c")
  sid = lax.axis_index("s")

  pltpu.sync_copy(zeros_hbm.at[pl.ds(0, ZROWS)],
                  acc.at[pl.ds(sid * ZROWS, ZROWS)])
  plsc.subcore_barrier()

  RG = 4
  RS = 2

  def run_region(region):
    fbase = region * RF
    pltpu.sync_copy(pcnt.at[cid * NW + region], cnt_v)
    n = cnt_v[pl.ds(0, LANES)][0] * CPF

    def issue_load(c):
      fl = fbase + lax.div(c, CPF)
      o = lax.rem(c, CPF) * CK
      pltpu.async_copy(ped.at[cid, fl, pl.ds(0, 3), pl.ds(o, CK)],
                       cidx.at[lax.rem(c, RI)], sem_l)

    def wait_load():
      pltpu.make_async_copy(ped.at[cid, fbase, pl.ds(0, 3), pl.ds(0, CK)],
                            cidx.at[0], sem_l).wait()

    def issue_gather(c):
      pltpu.async_copy(table_bf.at[cidx.at[lax.rem(c, RI), 0]],
                       rows_bf.at[lax.rem(c, RG)], sem_g)

    def wait_gather(c):
      pltpu.make_async_copy(table_bf.at[cidx.at[lax.rem(c, RI), 0]],
                            rows_bf.at[lax.rem(c, RG)], sem_g).wait()

    def issue_scatter(c):
      pltpu.async_copy(srows.at[lax.rem(c, RS)],
                       acc.at[cidx.at[lax.rem(c, RI), 1]], sem_s, add=True)

    def wait_scatter(c):
      pltpu.make_async_copy(srows.at[lax.rem(c, RS)],
                            acc.at[cidx.at[lax.rem(c, RI), 1]], sem_s).wait()

    def scale(c):
      b = lax.rem(c, RG)
      bs = lax.rem(c, RS)
      bw = lax.rem(c, RI)
      himask = jnp.int32(-65536)

      def scale_body(g, cc):
        w16 = plsc.bitcast(cidx[bw, 2, pl.ds(g * LANES, LANES)], jnp.float32)
        for j in range(LANES):
          wv = jnp.full((LANES,), w16[j], jnp.float32)
          e = g * LANES + j
          for k in range(2):
            wd = rows_bf[b, e, pl.ds(k * LANES, LANES)]
            lof = plsc.bitcast(wd << 16, jnp.float32)
            hif = plsc.bitcast(wd & himask, jnp.float32)
            srows[bs, e, pl.ds(k * LANES, LANES)] = lof * wv
            srows[bs, e, pl.ds((k + 2) * LANES, LANES)] = hif * wv
        return cc
      lax.fori_loop(0, CK // LANES, scale_body, 0)

    for k in range(5):
      issue_load(k)
    for k in range(3):
      wait_load()
      issue_gather(k)

    def step(c, carry):
      @pl.when(c >= 2)
      def _():
        wait_scatter(c - 2)

      @pl.when(c + 5 < n)
      def _():
        issue_load(c + 5)

      @pl.when(c + 3 < n)
      def _():
        wait_load()
        issue_gather(c + 3)
      wait_gather(c)
      scale(c)
      issue_scatter(c)
      return carry

    lax.fori_loop(0, n, step, 0)
    wait_scatter(n - 2)
    wait_scatter(n - 1)

  run_region(2 * sid)
  run_region(2 * sid + 1)
  plsc.subcore_barrier()

  base_node = cid * HALF
  ROWS_A = 1560
  pltpu.sync_copy(acc.at[pl.ds(sid * ROWS_A, ROWS_A)],
                  out.at[pl.ds(base_node + sid * ROWS_A, ROWS_A)])

  @pl.when(sid < (HALF - NS * ROWS_A) // 8)
  def _():
    pltpu.sync_copy(acc.at[pl.ds(NS * ROWS_A + sid * 8, 8)],
                    out.at[pl.ds(base_node + NS * ROWS_A + sid * 8, 8)])

  def pack_blk(i, c):
    r0 = sid * ROWS_A + i * 120
    pltpu.sync_copy(acc.at[pl.ds(r0, 120)], pf)
    _pack_rows(pf, pi, 120)
    pltpu.sync_copy(pi.at[pl.ds(0, 120)],
                    out_bf.at[pl.ds(base_node + r0, 120)])
    return c
  lax.fori_loop(0, ROWS_A // 120, pack_blk, 0)

  @pl.when(sid < (HALF - NS * ROWS_A) // 8)
  def _():
    r0 = NS * ROWS_A + sid * 8
    pltpu.sync_copy(acc.at[pl.ds(r0, 8)], pf.at[pl.ds(0, 8)])
    _pack_rows(pf, pi, 8)
    pltpu.sync_copy(pi.at[pl.ds(0, 8)], out_bf.at[pl.ds(base_node + r0, 8)])


_layer = pl.kernel(
    _layer_body,
    out_type=(
        jax.ShapeDtypeStruct((PAD_ROWS, DIM), jnp.float32),
        jax.ShapeDtypeStruct((PAD_ROWS, DIM // 2), jnp.int32),
    ),
    mesh=_mesh,
    compiler_params=_params,
    scratch_types=[
        pltpu.VMEM_SHARED((ACC_ROWS, DIM), jnp.float32),
        pltpu.VMEM((RI, 3, CK), jnp.int32),
        pltpu.VMEM((4, CK, DIM // 2), jnp.int32),
        pltpu.VMEM((2, CK, DIM), jnp.float32),
        pltpu.VMEM((120, DIM), jnp.float32),
        pltpu.VMEM((120, DIM // 2), jnp.int32),
        pltpu.VMEM((LANES,), jnp.int32),
        pltpu.SemaphoreType.DMA,
        pltpu.SemaphoreType.DMA,
        pltpu.SemaphoreType.DMA,
    ],
)


B_PER_TILE = BATCH // NW


def _combine_body(t0, t1, t2, t3, users, items, out,
                  uidx, iidx, urows, irows, gbuf, sem):
  cid = lax.axis_index("c")
  sid = lax.axis_index("s")
  wid = sid * NC + cid
  base = wid * B_PER_TILE

  pltpu.sync_copy(users.at[pl.ds(base, B_PER_TILE)], uidx.at[0])
  pltpu.sync_copy(items.at[pl.ds(base, B_PER_TILE)], iidx.at[0])
  for g in range(B_PER_TILE // LANES):
    iidx[0, pl.ds(g * LANES, LANES)] = (
        iidx[0, pl.ds(g * LANES, LANES)] + N_USERS)

  copies = []
  for k, t in enumerate((t0, t1, t2, t3)):
    copies.append(pltpu.async_copy(t.at[uidx.at[0]], urows.at[k], sem))
    copies.append(pltpu.async_copy(t.at[iidx.at[0]], irows.at[k], sem))
  for c in copies:
    c.wait()

  lanes = lax.iota(jnp.int32, LANES)

  def dot_body(g, c):
    res = jnp.zeros((LANES,), jnp.float32)
    for j in range(LANES):
      e = g * LANES + j
      p = jnp.zeros((LANES,), jnp.float32)
      for cb in range(DIM // LANES):
        sl = pl.ds(cb * LANES, LANES)
        us = (urows[0, e, sl] + urows[1, e, sl]
              + urows[2, e, sl] + urows[3, e, sl])
        vs = (irows[0, e, sl] + irows[1, e, sl]
              + irows[2, e, sl] + irows[3, e, sl])
        p = p + us * vs
      s = jnp.sum(p, axis=0) * jnp.float32(1.0 / ((N_LAYERS + 1) ** 2))
      res = jnp.where(lanes == j, jnp.full((LANES,), s, jnp.float32), res)
    gbuf[pl.ds(g * LANES, LANES)] = res
    return c
  lax.fori_loop(0, B_PER_TILE // LANES, dot_body, 0)

  pltpu.sync_copy(gbuf, out.at[pl.ds(base, B_PER_TILE)])


_combine = pl.kernel(
    _combine_body,
    out_type=jax.ShapeDtypeStruct((BATCH,), jnp.float32),
    mesh=_mesh,
    compiler_params=_params,
    scratch_types=[
        pltpu.VMEM((1, B_PER_TILE), jnp.int32),
        pltpu.VMEM((1, B_PER_TILE), jnp.int32),
        pltpu.VMEM((4, B_PER_TILE, DIM), jnp.float32),
        pltpu.VMEM((4, B_PER_TILE, DIM), jnp.float32),
        pltpu.VMEM((B_PER_TILE,), jnp.float32),
        pltpu.SemaphoreType.DMA,
    ],
)


@jax.jit
def kernel(users, items, edge_index, edge_weight, user_emb, item_emb):
  src = edge_index[0]
  dst = edge_index[1]
  pad = E_PAD - N_EDGES
  src_p = jnp.concatenate([src, jnp.zeros((pad,), jnp.int32)])
  dst_p = jnp.concatenate([dst, jnp.full((pad,), N_NODES, jnp.int32)])
  w_p = jnp.concatenate([edge_weight, jnp.zeros((pad,), jnp.float32)])
  zeros_hbm = jnp.zeros((ZROWS, DIM), jnp.float32)

  t0 = jnp.concatenate(
      [user_emb, item_emb, jnp.zeros((PAD_ROWS - N_NODES, DIM), jnp.float32)],
      axis=0)
  ped, pcnt, t0_bf = _partition(src_p, dst_p, w_p, t0)
  t1, t1_bf = _layer(t0_bf, ped, pcnt, zeros_hbm)
  t2, t2_bf = _layer(t1_bf, ped, pcnt, zeros_hbm)
  t3, _ = _layer(t2_bf, ped, pcnt, zeros_hbm)
  return _combine(t0, t1, t2, t3, users, items)
